# Initial kernel scaffold; baseline (speedup 1.0000x reference)
import jax
import jax.numpy as jnp
from jax.experimental import pallas as pl


def kernel(x_c, edge_index_c, batch_c, x_s, edge_index_s, batch_s, W_c0, b_c0, g_c0, be_c0, W_c1, b_c1, g_c1, be_c1, W_s0, b_s0, g_s0, be_s0, W_s1, b_s1, g_s1, be_s1, fc1_W, fc1_b, fc2_W, fc2_b):
    raise NotImplementedError("write your pallas kernel here")


# speedup vs baseline: 14.5867x; 14.5867x over previous
import functools

import jax
import jax.numpy as jnp
from jax import lax
from jax.experimental import pallas as pl
from jax.experimental.pallas import tpu as pltpu
from jax.experimental.pallas import tpu_sc as plsc

N = 100000
E = 1600000
EP = 1605632
G = 1024
H = 64
F16 = 16
NP = 100096
RS = NP // 16
ER = EP // 128
WR = 8
NWIN = ER // WR
TMAX = (NWIN + 31) // 32
NC, NS = 2, 16
BR = 3128
BRP = 256
EPS = 1e-5

@functools.lru_cache(maxsize=None)
def _mesh():
  return plsc.VectorSubcoreMesh(
      core_axis_name="c", subcore_axis_name="s",
      num_cores=NC, num_subcores=NS)


def _worker_id():
  return lax.axis_index("c") * NS + lax.axis_index("s")


def _edge_loop(src_hbm, dst_hbm, body):
  wid = _worker_id()

  def step(t, _):
    w = wid + 32 * t

    @pl.when(w < NWIN)
    def _():
      body(w)
    return 0

  lax.fori_loop(0, TMAX, step, 0)


@functools.lru_cache(maxsize=None)
def _sc_deg_kernel():
  return functools.partial(
      pl.kernel,
      out_type=jax.ShapeDtypeStruct((2, NC, NP), jnp.float32),
      mesh=_mesh(),
      scratch_types=[
          pltpu.VMEM((WR, 128), jnp.int32),
          pltpu.VMEM((WR * 128,), jnp.float32),
          pltpu.VMEM_SHARED((NP,), jnp.float32),
      ],
      compiler_params=pltpu.CompilerParams(use_tc_tiling_on_sc=False),
  )(_sc_deg_body)


def _sc_deg(*args):
  return _sc_deg_kernel()(*args)


def _sc_deg_body(dst_c, dst_s, z1, o1, out, idxv, onesv, acc):
  cid = lax.axis_index("c")
  sid = lax.axis_index("s")
  pltpu.sync_copy(o1, onesv)
  for br, dr in ((0, dst_c), (1, dst_s)):
    pltpu.sync_copy(z1, acc.at[pl.ds(sid * RS, RS)])
    plsc.subcore_barrier()

    def window(w):
      pltpu.sync_copy(dr.at[pl.ds(w * WR, WR), :], idxv)
      for j in range(WR):
        pltpu.sync_copy(onesv.at[pl.ds(j * 128, 128)],
                        acc.at[idxv.at[j]], add=True)

    _edge_loop(dr, dr, window)
    plsc.subcore_barrier()
    pltpu.sync_copy(acc.at[pl.ds(sid * RS, RS)],
                    out.at[br, cid, pl.ds(sid * RS, RS)])
    plsc.subcore_barrier()


@functools.lru_cache(maxsize=None)
def _sc_agg0_kernel():
  return functools.partial(
      pl.kernel,
      out_type=jax.ShapeDtypeStruct((2, NC, NP, F16), jnp.float32),
      mesh=_mesh(),
      scratch_types=[
          pltpu.VMEM((WR, 128), jnp.int32),
          pltpu.VMEM((WR, 128), jnp.int32),
          pltpu.VMEM((WR * 128, F16), jnp.float32),
          pltpu.VMEM_SHARED((NP, F16), jnp.float32),
          pltpu.SemaphoreType.DMA,
      ],
      compiler_params=pltpu.CompilerParams(use_tc_tiling_on_sc=False),
  )(_sc_agg0_body)


def _sc_agg0(*args):
  return _sc_agg0_kernel()(*args)


def _sc_agg0_body(src_c, dst_c, src_s, dst_s, xn_c, xn_s, z16, out,
                  srcv, dstv, rows, acc, sem):
  cid = lax.axis_index("c")
  sid = lax.axis_index("s")
  for br, (sr, dr, tbl) in ((0, (src_c, dst_c, xn_c)),
                            (1, (src_s, dst_s, xn_s))):
    pltpu.sync_copy(z16, acc.at[pl.ds(sid * RS, RS), :])
    plsc.subcore_barrier()

    def window(w, sr=sr, dr=dr, tbl=tbl):
      pltpu.sync_copy(sr.at[pl.ds(w * WR, WR), :], srcv)
      pltpu.sync_copy(dr.at[pl.ds(w * WR, WR), :], dstv)
      cps = [pltpu.async_copy(tbl.at[srcv.at[j]],
                              rows.at[pl.ds(j * 128, 128), :], sem)
             for j in range(WR)]
      for cp in cps:
        cp.wait()
      for j in range(WR):
        pltpu.sync_copy(rows.at[pl.ds(j * 128, 128), :],
                        acc.at[dstv.at[j]], add=True)

    _edge_loop(sr, dr, window)
    plsc.subcore_barrier()
    pltpu.sync_copy(acc.at[pl.ds(sid * RS, RS), :],
                    out.at[br, cid, pl.ds(sid * RS, RS), :])
    plsc.subcore_barrier()


_SEG_OUT = tuple(jax.ShapeDtypeStruct((NC, NP, F16), jnp.float32)
                 for _ in range(8))


@functools.lru_cache(maxsize=None)
def _sc_agg1_kernel():
  return functools.partial(
      pl.kernel,
      out_type=_SEG_OUT,
      mesh=_mesh(),
      scratch_types=[
          pltpu.VMEM((WR, 128), jnp.int32),
          pltpu.VMEM((WR, 128), jnp.int32),
          pltpu.VMEM((WR * 128, F16), jnp.float32),
          pltpu.VMEM_SHARED((NP, F16), jnp.float32),
          pltpu.SemaphoreType.DMA,
      ],
      compiler_params=pltpu.CompilerParams(use_tc_tiling_on_sc=False),
  )(_sc_agg1_body)


def _sc_agg1(*args):
  return _sc_agg1_kernel()(*args)


def _sc_agg1_body(src_c, dst_c, src_s, dst_s,
                  hc0, hc1, hc2, hc3, hs0, hs1, hs2, hs3, z16,
                  o0, o1, o2, o3, o4, o5, o6, o7,
                  srcv, dstv, rows, acc, sem):
  cid = lax.axis_index("c")
  sid = lax.axis_index("s")
  segs = ((src_c, dst_c, hc0, o0), (src_c, dst_c, hc1, o1),
          (src_c, dst_c, hc2, o2), (src_c, dst_c, hc3, o3),
          (src_s, dst_s, hs0, o4), (src_s, dst_s, hs1, o5),
          (src_s, dst_s, hs2, o6), (src_s, dst_s, hs3, o7))
  for sr, dr, tbl, ob in segs:
    pltpu.sync_copy(z16, acc.at[pl.ds(sid * RS, RS), :])
    plsc.subcore_barrier()

    def window(w, sr=sr, dr=dr, tbl=tbl):
      pltpu.sync_copy(sr.at[pl.ds(w * WR, WR), :], srcv)
      pltpu.sync_copy(dr.at[pl.ds(w * WR, WR), :], dstv)
      cps = [pltpu.async_copy(tbl.at[srcv.at[j]],
                              rows.at[pl.ds(j * 128, 128), :], sem)
             for j in range(WR)]
      for cp in cps:
        cp.wait()
      for j in range(WR):
        pltpu.sync_copy(rows.at[pl.ds(j * 128, 128), :],
                        acc.at[dstv.at[j]], add=True)

    _edge_loop(sr, dr, window)
    plsc.subcore_barrier()
    pltpu.sync_copy(acc.at[pl.ds(sid * RS, RS), :],
                    ob.at[cid, pl.ds(sid * RS, RS), :])
    plsc.subcore_barrier()


def _dinv_body(d_ref, o_ref):
  o_ref[0] = lax.rsqrt(1.0 + d_ref[0, 0] + d_ref[0, 1])


def _tc_dinv(degp):
  return pl.pallas_call(
      _dinv_body,
      grid=(2,),
      in_specs=[pl.BlockSpec((1, 2, NP // 128, 128), lambda b: (b, 0, 0, 0))],
      out_specs=pl.BlockSpec((1, NP // 128, 128), lambda b: (b, 0, 0)),
      out_shape=jax.ShapeDtypeStruct((2, NP // 128, 128), jnp.float32),
  )(degp)


def _xn_body(x_ref, d_ref, o_ref):
  o_ref[...] = x_ref[...] * d_ref[...]


def _tc_xn(x_pad, dinv_col):
  return pl.pallas_call(
      _xn_body,
      grid=(NP // BR,),
      in_specs=[pl.BlockSpec((BR, F16), lambda i: (i, 0)),
                pl.BlockSpec((BR, 1), lambda i: (i, 0))],
      out_specs=pl.BlockSpec((BR, F16), lambda i: (i, 0)),
      out_shape=jax.ShapeDtypeStruct((NP, F16), jnp.float32),
  )(x_pad, dinv_col)


def _row_mask(i, rows):
  ridx = i * rows + lax.broadcasted_iota(jnp.int32, (rows, 1), 0)
  return (ridx < N).astype(jnp.float32)


def _l0_body(acc_ref, xn_ref, d_ref, w_ref, b_ref, o_ref, st_ref):
  i = pl.program_id(0)
  a = acc_ref[0] + acc_ref[1] + xn_ref[...]
  h = jnp.dot(a, w_ref[...], preferred_element_type=jnp.float32)
  h = h * d_ref[...] + b_ref[...]
  o_ref[...] = h

  @pl.when(i == 0)
  def _():
    st_ref[...] = jnp.zeros_like(st_ref)

  m = _row_mask(i, BR)
  hm = h * m
  s1 = jnp.sum(hm, axis=0)[None]
  s2 = jnp.sum(hm * h, axis=0)[None]
  st_ref[...] += jnp.concatenate(
      [s1, s2, jnp.zeros((6, H), jnp.float32)], axis=0)


def _tc_l0(accp, xn, dinv_col, w_pad, bias):
  return pl.pallas_call(
      _l0_body,
      grid=(NP // BR,),
      in_specs=[pl.BlockSpec((2, BR, F16), lambda i: (0, i, 0)),
                pl.BlockSpec((BR, F16), lambda i: (i, 0)),
                pl.BlockSpec((BR, 1), lambda i: (i, 0)),
                pl.BlockSpec((F16, H), lambda i: (0, 0)),
                pl.BlockSpec((1, H), lambda i: (0, 0))],
      out_specs=[pl.BlockSpec((BR, H), lambda i: (i, 0)),
                 pl.BlockSpec((8, H), lambda i: (0, 0))],
      out_shape=[jax.ShapeDtypeStruct((NP, H), jnp.float32),
                 jax.ShapeDtypeStruct((8, H), jnp.float32)],
  )(accp, xn, dinv_col, w_pad, bias)


def _bn_relu(h, st_ref, g_ref, be_ref):
  mean = st_ref[0, :][None] / N
  var = st_ref[1, :][None] / N - mean * mean
  scale = g_ref[...] * lax.rsqrt(var + EPS)
  return jnp.maximum((h - mean) * scale + be_ref[...], 0.0)


def _l0b_body(h_ref, st_ref, g_ref, be_ref, w_ref, d_ref, o_ref):
  x = _bn_relu(h_ref[...], st_ref, g_ref, be_ref)
  hn = jnp.dot(x, w_ref[...], preferred_element_type=jnp.float32)
  hn = hn * d_ref[...]
  for c in range(4):
    o_ref[c] = hn[:, c * F16:(c + 1) * F16]


def _tc_l0b(h0pre, st, gam, bet, w1, dinv_col):
  return pl.pallas_call(
      _l0b_body,
      grid=(NP // BR,),
      in_specs=[pl.BlockSpec((BR, H), lambda i: (i, 0)),
                pl.BlockSpec((8, H), lambda i: (0, 0)),
                pl.BlockSpec((1, H), lambda i: (0, 0)),
                pl.BlockSpec((1, H), lambda i: (0, 0)),
                pl.BlockSpec((H, H), lambda i: (0, 0)),
                pl.BlockSpec((BR, 1), lambda i: (i, 0))],
      out_specs=pl.BlockSpec((4, BR, F16), lambda i: (0, i, 0)),
      out_shape=jax.ShapeDtypeStruct((4, NP, F16), jnp.float32),
  )(h0pre, st, gam, bet, w1, dinv_col)


def _l1_body(p0_ref, p1_ref, p2_ref, p3_ref, hn_ref, d_ref, b_ref,
             o_ref, st_ref):
  i = pl.program_id(0)
  cols = []
  for c, p in enumerate((p0_ref, p1_ref, p2_ref, p3_ref)):
    cols.append(p[0] + p[1] + hn_ref[c])
  h = jnp.concatenate(cols, axis=1) * d_ref[...] + b_ref[...]
  o_ref[...] = h

  @pl.when(i == 0)
  def _():
    st_ref[...] = jnp.zeros_like(st_ref)

  m = _row_mask(i, BR)
  hm = h * m
  s1 = jnp.sum(hm, axis=0)[None]
  s2 = jnp.sum(hm * h, axis=0)[None]
  st_ref[...] += jnp.concatenate(
      [s1, s2, jnp.zeros((6, H), jnp.float32)], axis=0)


def _tc_l1(parts, hn1, dinv_col, bias):
  pspec = pl.BlockSpec((2, BR, F16), lambda i: (0, i, 0))
  return pl.pallas_call(
      _l1_body,
      grid=(NP // BR,),
      in_specs=[pspec, pspec, pspec, pspec,
                pl.BlockSpec((4, BR, F16), lambda i: (0, i, 0)),
                pl.BlockSpec((BR, 1), lambda i: (i, 0)),
                pl.BlockSpec((1, H), lambda i: (0, 0))],
      out_specs=[pl.BlockSpec((BR, H), lambda i: (i, 0)),
                 pl.BlockSpec((8, H), lambda i: (0, 0))],
      out_shape=[jax.ShapeDtypeStruct((NP, H), jnp.float32),
                 jax.ShapeDtypeStruct((8, H), jnp.float32)],
  )(parts[0], parts[1], parts[2], parts[3], hn1, dinv_col, bias)


def _pool_body(h_ref, st_ref, g_ref, be_ref, b_ref, ps_ref, cnt_ref):
  i = pl.program_id(0)
  x = _bn_relu(h_ref[...], st_ref, g_ref, be_ref)
  bidx = b_ref[...]
  oh = (bidx == lax.broadcasted_iota(jnp.int32, (BRP, G), 1)).astype(
      jnp.float32)

  @pl.when(i == 0)
  def _():
    ps_ref[...] = jnp.zeros_like(ps_ref)
    cnt_ref[...] = jnp.zeros_like(cnt_ref)

  ps_ref[...] += lax.dot_general(oh, x, (((0,), (0,)), ((), ())),
                                 preferred_element_type=jnp.float32)
  cnt_ref[...] += jnp.sum(oh, axis=0, keepdims=True)


def _tc_pool(h1pre, st, gam, bet, batch_col):
  return pl.pallas_call(
      _pool_body,
      grid=(NP // BRP,),
      in_specs=[pl.BlockSpec((BRP, H), lambda i: (i, 0)),
                pl.BlockSpec((8, H), lambda i: (0, 0)),
                pl.BlockSpec((1, H), lambda i: (0, 0)),
                pl.BlockSpec((1, H), lambda i: (0, 0)),
                pl.BlockSpec((BRP, 1), lambda i: (i, 0))],
      out_specs=[pl.BlockSpec((G, H), lambda i: (0, 0)),
                 pl.BlockSpec((1, G), lambda i: (0, 0))],
      out_shape=[jax.ShapeDtypeStruct((G, H), jnp.float32),
                 jax.ShapeDtypeStruct((1, G), jnp.float32)],
  )(h1pre, st, gam, bet, batch_col)


def _head_body(pc_ref, cc_ref, ps_ref, cs_ref, wc_ref, ws_ref, b1_ref,
               w2_ref, b2_ref, o_ref):
  pc = pc_ref[...] / jnp.maximum(cc_ref[...], 1.0)
  ps = ps_ref[...] / jnp.maximum(cs_ref[...], 1.0)
  z = (jnp.dot(pc, wc_ref[...], preferred_element_type=jnp.float32)
       + jnp.dot(ps, ws_ref[...], preferred_element_type=jnp.float32)
       + b1_ref[...])
  z = jnp.maximum(z, 0.0)
  o_ref[...] = jnp.dot(z, w2_ref[...],
                       preferred_element_type=jnp.float32) + b2_ref[...]


def _tc_head(psum_c, cnt_c, psum_s, cnt_s, wc, ws, b1, w2p, b2p):
  return pl.pallas_call(
      _head_body,
      out_shape=jax.ShapeDtypeStruct((G, 128), jnp.float32),
  )(psum_c, cnt_c, psum_s, cnt_s, wc, ws, b1, w2p, b2p)


def kernel(x_c, edge_index_c, batch_c, x_s, edge_index_s, batch_s,
           W_c0, b_c0, g_c0, be_c0, W_c1, b_c1, g_c1, be_c1,
           W_s0, b_s0, g_s0, be_s0, W_s1, b_s1, g_s1, be_s1,
           fc1_W, fc1_b, fc2_W, fc2_b):
  f32 = jnp.float32
  def _edges(ei):
    pad = jnp.full((2, EP - E), N, dtype=ei.dtype)
    return jnp.concatenate([ei, pad], axis=1).reshape(2, ER, 128)

  ec = _edges(edge_index_c)
  es = _edges(edge_index_s)
  src_c, dst_c = ec[0], ec[1]
  src_s, dst_s = es[0], es[1]
  z1 = jnp.zeros((RS,), f32)
  z16 = jnp.zeros((RS, F16), f32)
  o1 = jnp.ones((WR * 128,), f32)

  degp = _sc_deg(dst_c, dst_s, z1, o1)
  dinv2 = _tc_dinv(degp.reshape(2, 2, NP // 128, 128))
  dinv_c = dinv2[0].reshape(NP, 1)
  dinv_s = dinv2[1].reshape(NP, 1)

  xp_c = jnp.pad(x_c, ((0, NP - N), (0, F16 - x_c.shape[1])))
  xp_s = jnp.pad(x_s, ((0, NP - N), (0, F16 - x_s.shape[1])))
  xn_c = _tc_xn(xp_c, dinv_c)
  xn_s = _tc_xn(xp_s, dinv_s)

  accx = _sc_agg0(src_c, dst_c, src_s, dst_s, xn_c, xn_s, z16)

  w0c = jnp.pad(W_c0, ((0, F16 - W_c0.shape[0]), (0, 0)))
  w0s = jnp.pad(W_s0, ((0, F16 - W_s0.shape[0]), (0, 0)))
  h0_c, st0_c = _tc_l0(accx[0], xn_c, dinv_c, w0c, b_c0.reshape(1, H))
  h0_s, st0_s = _tc_l0(accx[1], xn_s, dinv_s, w0s, b_s0.reshape(1, H))

  hn1_c = _tc_l0b(h0_c, st0_c, g_c0.reshape(1, H), be_c0.reshape(1, H),
                  W_c1, dinv_c)
  hn1_s = _tc_l0b(h0_s, st0_s, g_s0.reshape(1, H), be_s0.reshape(1, H),
                  W_s1, dinv_s)

  acc1 = _sc_agg1(src_c, dst_c, src_s, dst_s,
                  hn1_c[0], hn1_c[1], hn1_c[2], hn1_c[3],
                  hn1_s[0], hn1_s[1], hn1_s[2], hn1_s[3], z16)

  h1_c, st1_c = _tc_l1(acc1[0:4], hn1_c, dinv_c, b_c1.reshape(1, H))
  h1_s, st1_s = _tc_l1(acc1[4:8], hn1_s, dinv_s, b_s1.reshape(1, H))

  bp_c = jnp.pad(batch_c, (0, NP - N), constant_values=G).reshape(NP, 1)
  bp_s = jnp.pad(batch_s, (0, NP - N), constant_values=G).reshape(NP, 1)
  psum_c, cnt_c = _tc_pool(h1_c, st1_c, g_c1.reshape(1, H),
                           be_c1.reshape(1, H), bp_c)
  psum_s, cnt_s = _tc_pool(h1_s, st1_s, g_s1.reshape(1, H),
                           be_s1.reshape(1, H), bp_s)

  w2p = jnp.pad(fc2_W, ((0, 0), (0, 128 - fc2_W.shape[1])))
  b2p = jnp.pad(fc2_b, (0, 128 - fc2_b.shape[0])).reshape(1, 128)
  out = _tc_head(psum_c, cnt_c.reshape(G, 1), psum_s, cnt_s.reshape(G, 1),
                 fc1_W[:H], fc1_W[H:], fc1_b.reshape(1, H), w2p, b2p)
  return out[:, :fc2_W.shape[1]]
